# baseline (speedup 1.0000x reference)
import jax
import jax.numpy as jnp
from jax.experimental import pallas as pl
from jax.experimental.pallas import tpu as pltpu


def _sp_body(enc_ref, mask_ref, out_ref):
    m = mask_ref[:, 0]
    e = enc_ref[...]
    repl = jnp.float32(3.0) - jnp.float32(2.0) * m.astype(jnp.float32)
    out_ref[...] = jnp.where((m == 0)[:, None], e, repl[:, None])


def _run_range(encoded, mask, b0, nb):
    b, c, h, w = encoded.shape
    B = 2
    grid = (nb // B,)
    return pl.pallas_call(
        _sp_body,
        grid=grid,
        in_specs=[
            pl.BlockSpec((B, c, h, w), lambda i: (b0 // B + i, 0, 0, 0)),
            pl.BlockSpec((B, 1, h, w), lambda i: (b0 // B + i, 0, 0, 0)),
        ],
        out_specs=pl.BlockSpec((B, c, h, w), lambda i: (i, 0, 0, 0)),
        out_shape=jax.ShapeDtypeStruct((nb, c, h, w), encoded.dtype),
    )(encoded, mask)


def kernel(encoded, cover_img, mask):
    out_a = _run_range(encoded, mask, 0, 8)
    out_b = _run_range(encoded, mask, 8, 8)
    return jnp.concatenate([out_a, out_b], axis=0)
